# Initial kernel scaffold; baseline (speedup 1.0000x reference)
import jax
import jax.numpy as jnp
from jax.experimental import pallas as pl


def kernel(hidden_states, W_gate, w1, w3, w2):
    raise NotImplementedError("write your pallas kernel here")


# speedup vs baseline: 1.6661x; 1.6661x over previous
import functools

import jax
import jax.numpy as jnp
from jax.experimental import pallas as pl
from jax.experimental.pallas import tpu as pltpu

_E = 8
_H = 2048
_I = 1024
_EPAD = 128


def _router_body(x_ref, wg_ref, dw_ref):
    logits = jnp.dot(x_ref[...], wg_ref[...], preferred_element_type=jnp.float32)
    col = jax.lax.broadcasted_iota(jnp.int32, logits.shape, 1)
    masked = jnp.where(col < _E, logits, jnp.float32(-1e30))
    m = jnp.max(masked, axis=1, keepdims=True)
    p = jnp.exp(masked - m)
    p = p / jnp.sum(p, axis=1, keepdims=True)
    w1v = jnp.max(p, axis=1, keepdims=True)
    i1 = jnp.argmax(p, axis=1)[:, None]
    p2 = jnp.where(col == i1, jnp.float32(-1.0), p)
    w2v = jnp.max(p2, axis=1, keepdims=True)
    i2 = jnp.argmax(p2, axis=1)[:, None]
    s = w1v + w2v
    dw_ref[...] = jnp.where(col == i1, w1v / s, 0.0) + jnp.where(
        col == i2, w2v / s, 0.0)


def _moe_body(x_ref, dw_ref, w1_ref, w3_ref, w2_ref, out_ref):
    e = pl.program_id(1)
    i = pl.program_id(2)
    x = x_ref[...]
    g = jnp.dot(x, w1_ref[0], preferred_element_type=jnp.float32)
    u = jnp.dot(x, w3_ref[0], preferred_element_type=jnp.float32)
    h = (g * jax.nn.sigmoid(g)) * u
    col = jax.lax.broadcasted_iota(jnp.int32, dw_ref.shape, 1)
    scale = jnp.sum(jnp.where(col == e, dw_ref[...], 0.0), axis=1, keepdims=True)
    o = jnp.dot(h * scale, w2_ref[0], preferred_element_type=jnp.float32)

    @pl.when((e == 0) & (i == 0))
    def _():
        out_ref[...] = o

    @pl.when((e > 0) | (i > 0))
    def _():
        out_ref[...] += o


def kernel(hidden_states, W_gate, w1, w3, w2):
    orig_shape = hidden_states.shape
    x = hidden_states.reshape(-1, _H)
    t = x.shape[0]

    wg_pad = jnp.zeros((_H, _EPAD), jnp.float32).at[:, :_E].set(W_gate)

    bt_r = 512
    dw = pl.pallas_call(
        _router_body,
        grid=(t // bt_r,),
        in_specs=[
            pl.BlockSpec((bt_r, _H), lambda i: (i, 0)),
            pl.BlockSpec((_H, _EPAD), lambda i: (0, 0)),
        ],
        out_specs=pl.BlockSpec((bt_r, _EPAD), lambda i: (i, 0)),
        out_shape=jax.ShapeDtypeStruct((t, _EPAD), jnp.float32),
    )(x, wg_pad)

    bt = 1024
    ti = 256
    out = pl.pallas_call(
        _moe_body,
        grid=(t // bt, _E, _I // ti),
        in_specs=[
            pl.BlockSpec((bt, _H), lambda b, e, i: (b, 0)),
            pl.BlockSpec((bt, _EPAD), lambda b, e, i: (b, 0)),
            pl.BlockSpec((1, _H, ti), lambda b, e, i: (e, 0, i)),
            pl.BlockSpec((1, _H, ti), lambda b, e, i: (e, 0, i)),
            pl.BlockSpec((1, ti, _H), lambda b, e, i: (e, i, 0)),
        ],
        out_specs=pl.BlockSpec((bt, _H), lambda b, e, i: (b, 0)),
        out_shape=jax.ShapeDtypeStruct((t, _H), jnp.float32),
        compiler_params=pltpu.CompilerParams(
            dimension_semantics=("arbitrary", "arbitrary", "arbitrary")),
    )(x, dw, w1, w3, w2)

    return out.reshape(orig_shape)
